# baseline (speedup 1.0000x reference)
import jax
import jax.numpy as jnp
from jax.experimental import pallas as pl


def _concat_body(pp, cp, sp, pn, cn, sn, tp, tn, out_p, out_t):
    out_p[0:1, :] = pp[...]
    out_p[1:13, :] = cp[...]
    out_p[13:77, :] = sp[...]
    out_p[77:78, :] = pn[...]
    out_p[78:90, :] = cn[...]
    out_p[90:154, :] = sn[...]
    out_t[0:1, :] = tp[...]
    out_t[1:2, :] = tn[...]


def kernel(ctx_pos, ctx_neg, token_prefix_pos, token_suffix_pos,
           token_prefix_neg, token_suffix_neg,
           tokenized_prompts_pos, tokenized_prompts_neg, cls_id):
    n_ctx = ctx_pos.shape[2]
    dim = ctx_pos.shape[3]
    suf = token_suffix_pos.shape[2]
    ctx_len = 1 + n_ctx + suf
    pp = token_prefix_pos.reshape(1, dim)
    cp = ctx_pos.reshape(n_ctx, dim)
    sp = token_suffix_pos.reshape(suf, dim)
    pn = token_prefix_neg.reshape(1, dim)
    cn = ctx_neg.reshape(n_ctx, dim)
    sn = token_suffix_neg.reshape(suf, dim)
    tp = tokenized_prompts_pos.reshape(1, ctx_len)
    tn = tokenized_prompts_neg.reshape(1, ctx_len)

    out_p, out_t = pl.pallas_call(
        _concat_body,
        out_shape=(
            jax.ShapeDtypeStruct((2 * ctx_len, dim), jnp.float32),
            jax.ShapeDtypeStruct((2, ctx_len), jnp.int32),
        ),
    )(pp, cp, sp, pn, cn, sn, tp, tn)
    return out_p.reshape(2, ctx_len, dim), out_t
